# Initial kernel scaffold; baseline (speedup 1.0000x reference)
import jax
import jax.numpy as jnp
from jax.experimental import pallas as pl


def kernel(time_segs, ee_W0, ee_b0, ee_W1, ee_b1, ne_W0, ne_b0, ne_W1, ne_b1, nd_W0, nd_b0, nd_W1, nd_b1, out_W, out_b):
    raise NotImplementedError("write your pallas kernel here")


# speedup vs baseline: 205.3410x; 205.3410x over previous
import jax
import jax.numpy as jnp
from jax.experimental import pallas as pl

_B, _N, _D, _H = 8, 256, 4, 32
_STEPS = 8
_PACK = 4
_HP = _H * _PACK
_GCH = 64
_NCH = _N // _GCH
_GP = _GCH // _PACK


def _fused_kernel(x_ref, Sperm_ref, W0a_ref, W0b_ref, eeb0_ref, eeW1_ref, W1blk_ref,
                  eeb1_ref, eeb1t_ref, J_ref,
                  neW0_ref, neb0_ref, neW1_ref, neb1_ref,
                  ndW0a_ref, ndW0b_ref, ndb0_ref, ndW1_ref, ndb1_ref,
                  outW_ref, outb_ref, out_ref):
    Sperm = Sperm_ref[:]
    W0a = W0a_ref[:]
    W0b = W0b_ref[:]
    eeb0 = eeb0_ref[:]
    eeW1 = eeW1_ref[:]
    W1blk = W1blk_ref[:]
    eeb1 = eeb1_ref[:]
    eeb1t = eeb1t_ref[:]
    J = J_ref[:]
    neW0 = neW0_ref[:]; neb0 = neb0_ref[:]
    neW1 = neW1_ref[:]; neb1 = neb1_ref[:]
    ndW0a = ndW0a_ref[:]
    ndW0b = ndW0b_ref[:]
    ndb0 = ndb0_ref[:]
    ndW1 = ndW1_ref[:]; ndb1 = ndb1_ref[:]
    outW = outW_ref[:]; outb = outb_ref[:]

    def dot(a, b):
        return jax.lax.dot_general(a, b, (((1,), (0,)), ((), ())),
                                   preferred_element_type=jnp.float32)

    def step_fn(i, x):
        U = dot(x, W0a)
        V = dot(x, W0b) + eeb0
        aggs = []
        for b in range(_B):
            Ub = U[b * _N:(b + 1) * _N, :]
            Vb = V[b * _N:(b + 1) * _N, :]
            P = dot(Sperm, Ub)
            Q = _N // _PACK
            Up = jnp.concatenate(
                [P[0:Q], P[Q:2 * Q], P[2 * Q:3 * Q], P[3 * Q:4 * Q]], axis=1)
            Vt = jnp.concatenate([Vb, Vb, Vb, Vb], axis=1)

            acc = jnp.zeros((_N, _H), jnp.float32)
            for c in range(_NCH):
                upc = Up[c * _GP:(c + 1) * _GP, :]
                a = upc[:, None, :] + Vt[None, :, :]
                h1 = jnp.maximum(a, 0.0).reshape(_GP * _N, _HP)
                z = dot(h1, W1blk) + eeb1t
                h2 = jnp.maximum(z, 0.0)
                s = dot(h2, J).reshape(_GP, _N, _H)
                acc = acc + jnp.sum(s, axis=0)
            aggs.append(acc)
        agg = jnp.concatenate(aggs, axis=0)
        dmsg = jnp.maximum(dot(jnp.maximum(U + V, 0.0), eeW1) + eeb1, 0.0)
        agg = agg - dmsg
        nm = jnp.maximum(dot(agg, neW0) + neb0, 0.0)
        nm = jnp.maximum(dot(nm, neW1) + neb1, 0.0)
        h = jnp.maximum(dot(x, ndW0a) + dot(nm, ndW0b) + ndb0, 0.0)
        h = jnp.maximum(dot(h, ndW1) + ndb1, 0.0)
        nxt = dot(h, outW) + outb + x
        out_ref[i, :, :] = nxt
        return nxt

    jax.lax.fori_loop(0, _STEPS, step_fn, x_ref[:])


def kernel(time_segs, ee_W0, ee_b0, ee_W1, ee_b1, ne_W0, ne_b0, ne_W1, ne_b1,
           nd_W0, nd_b0, nd_W1, nd_b1, out_W, out_b):
    x0 = time_segs.reshape(_B * _N, _D)
    W1blk = jnp.kron(jnp.eye(_PACK, dtype=jnp.float32), ee_W1)
    J = jnp.tile(jnp.eye(_H, dtype=jnp.float32), (_PACK, 1))
    q = _N // _PACK
    si = jnp.arange(_N) // q
    g = jnp.arange(_N) % q
    Sperm = jnp.zeros((_N, _N), jnp.float32).at[jnp.arange(_N), _PACK * g + si].set(1.0)
    out = pl.pallas_call(
        _fused_kernel,
        out_shape=jax.ShapeDtypeStruct((_STEPS, _B * _N, _D), jnp.float32),
    )(
        x0, Sperm,
        ee_W0[0:_D, :], ee_W0[_D:2 * _D, :], ee_b0[None, :],
        ee_W1, W1blk, ee_b1[None, :], jnp.tile(ee_b1, _PACK)[None, :], J,
        ne_W0, ne_b0[None, :], ne_W1, ne_b1[None, :],
        nd_W0[0:_D, :], nd_W0[_D:_D + _H, :], nd_b0[None, :],
        nd_W1, nd_b1[None, :],
        out_W, out_b[None, :],
    )
    return out.reshape(_STEPS, _B, _N, _D).transpose(1, 0, 2, 3)


# speedup vs baseline: 223.3765x; 1.0878x over previous
import jax
import jax.numpy as jnp
from jax.experimental import pallas as pl
from jax.experimental.pallas import tpu as pltpu

_B, _N, _D, _H = 8, 256, 4, 32
_STEPS = 8
_GRP = 4
_NG = _B // _GRP
_HP = _H * _GRP
_SC = 32
_NCH = _N // _SC


def _fused_kernel(x_ref, M_ref, W0blk_ref, eeb0t_ref, W1blk_ref, eeb1t_ref,
                  neW0blk_ref, neb0t_ref, neW1blk_ref, neb1t_ref,
                  ndW0blk_ref, ndb0t_ref, ndW1blk_ref, ndb1t_ref,
                  outWblk_ref, outbt_ref, out_ref, scr_ref):
    M = M_ref[:]
    W0blk = W0blk_ref[:]
    eeb0t = eeb0t_ref[:]
    W1blk = W1blk_ref[:]
    eeb1t = eeb1t_ref[:]
    neW0blk = neW0blk_ref[:]; neb0t = neb0t_ref[:]
    neW1blk = neW1blk_ref[:]; neb1t = neb1t_ref[:]
    ndW0blk = ndW0blk_ref[:]
    ndb0t = ndb0t_ref[:]
    ndW1blk = ndW1blk_ref[:]; ndb1t = ndb1t_ref[:]
    outWblk = outWblk_ref[:]
    outbt = outbt_ref[:]

    def dot(a, b):
        return jax.lax.dot_general(a, b, (((1,), (0,)), ((), ())),
                                   preferred_element_type=jnp.float32)

    zc = jnp.zeros((_N, _D), jnp.float32)

    def step_fn(i, xs_all):
        new_states = []
        for grp in range(_NG):
            xb = xs_all[grp * _GRP:(grp + 1) * _GRP]
            xs = jnp.concatenate(
                [v for b in range(_GRP) for v in (xb[b], zc)], axis=1)
            xt = jnp.concatenate(
                [v for b in range(_GRP) for v in (zc, xb[b])], axis=1)

            acc = jnp.zeros((_N, _HP), jnp.float32)
            for c in range(_NCH):
                xsc = xs[c * _SC:(c + 1) * _SC, :]
                a = xsc[:, None, :] + xt[None, :, :]
                h1 = jnp.maximum(
                    dot(a.reshape(_SC * _N, 2 * _D * _GRP), W0blk) + eeb0t, 0.0)
                h2 = jnp.maximum(dot(h1, W1blk) + eeb1t, 0.0)
                h2 = h2.reshape(_SC, _N, _HP)
                scr_ref[:, :, :] = h2 * M[c * _SC:(c + 1) * _SC, :, None]

                def chain(s, acc):
                    return acc + scr_ref[s]

                acc = jax.lax.fori_loop(0, _SC, chain, acc, unroll=True)
            nm = jnp.maximum(dot(acc, neW0blk) + neb0t, 0.0)
            nm = jnp.maximum(dot(nm, neW1blk) + neb1t, 0.0)
            d2 = jnp.concatenate(
                [v for b in range(_GRP)
                 for v in (xb[b], nm[:, b * _H:(b + 1) * _H])], axis=1)
            h = jnp.maximum(dot(d2, ndW0blk) + ndb0t, 0.0)
            h = jnp.maximum(dot(h, ndW1blk) + ndb1t, 0.0)
            xpk = jnp.concatenate(xb, axis=1)
            nxt = dot(h, outWblk) + outbt + xpk
            out_ref[i, grp] = nxt
            for b in range(_GRP):
                new_states.append(nxt[:, b * _D:(b + 1) * _D])
        return tuple(new_states)

    x0 = tuple(x_ref[b * _N:(b + 1) * _N, :] for b in range(_B))
    jax.lax.fori_loop(0, _STEPS, step_fn, x0)


def kernel(time_segs, ee_W0, ee_b0, ee_W1, ee_b1, ne_W0, ne_b0, ne_W1, ne_b1,
           nd_W0, nd_b0, nd_W1, nd_b1, out_W, out_b):
    x0 = time_segs.reshape(_B * _N, _D)
    eye = jnp.eye(_GRP, dtype=jnp.float32)
    M = jnp.ones((_N, _N), jnp.float32) - jnp.eye(_N, dtype=jnp.float32)
    out = pl.pallas_call(
        _fused_kernel,
        out_shape=jax.ShapeDtypeStruct((_STEPS, _NG, _N, _GRP * _D), jnp.float32),
        scratch_shapes=[pltpu.VMEM((_SC, _N, _HP), jnp.float32)],
    )(
        x0, M,
        jnp.kron(eye, ee_W0), jnp.tile(ee_b0, _GRP)[None, :],
        jnp.kron(eye, ee_W1), jnp.tile(ee_b1, _GRP)[None, :],
        jnp.kron(eye, ne_W0), jnp.tile(ne_b0, _GRP)[None, :],
        jnp.kron(eye, ne_W1), jnp.tile(ne_b1, _GRP)[None, :],
        jnp.kron(eye, nd_W0), jnp.tile(nd_b0, _GRP)[None, :],
        jnp.kron(eye, nd_W1), jnp.tile(nd_b1, _GRP)[None, :],
        jnp.kron(eye, out_W), jnp.tile(out_b, _GRP)[None, :],
    )
    out = out.reshape(_STEPS, _NG, _N, _GRP, _D)
    out = out.transpose(1, 3, 0, 2, 4).reshape(_B, _STEPS, _N, _D)
    return out
